# baseline (speedup 1.0000x reference)
import functools

import jax
import jax.numpy as jnp
from jax import lax
from jax.experimental import pallas as pl
from jax.experimental.pallas import tpu as pltpu
from jax.experimental.pallas import tpu_sc as plsc

B = 16384
D = 64
NC = 2
NS = 16
L = 16
NW = NC * NS
BPW = B // NW
CHUNK = 128
NCHUNK = BPW // CHUNK
NPH = 8
DEPTH = 3
HD = D // 2
NSG = BPW // L


def _body(uid_hbm, iid_hbm, uemb_hbm, iemb_hbm,
          out_hbm,
          uid_v, iid_v, ubufs, ibufs, out_v, sem_u, sem_i):
    wid = lax.axis_index("s") * NC + lax.axis_index("c")
    base = wid * BPW

    for c in range(NCHUNK):
        src = pl.ds(base + c * CHUNK, CHUNK)
        pltpu.sync_copy(uid_hbm.at[src], uid_v.at[c])
        pltpu.sync_copy(iid_hbm.at[src], iid_v.at[c])

    iota16 = lax.iota(jnp.int32, L)
    zero16 = jnp.zeros((L,), jnp.float32)

    def load_ids(sg):
        c = sg // (CHUNK // L)
        off = (sg % (CHUNK // L)) * L
        return c, pl.ds(off, L)

    def fire(vec_u, vec_i, lane, ph):
        su = vec_u[lane]
        si = vec_i[lane]
        su0 = pl.multiple_of((su >> 7) * 128, 128)
        si0 = pl.multiple_of((si >> 7) * 128, 128)
        for h in range(2):
            rsl = pl.ds(h * HD, HD)
            pltpu.async_copy(uemb_hbm.at[rsl, pl.ds(su0, 128)],
                             ubufs.at[(ph + h) % NPH], sem_u)
            pltpu.async_copy(iemb_hbm.at[rsl, pl.ds(si0, 128)],
                             ibufs.at[(ph + h) % NPH], sem_i)

    def drain(ph):
        for h in range(2):
            pltpu.make_async_copy(uemb_hbm.at[pl.ds(0, HD), pl.ds(0, 128)],
                                  ubufs.at[(ph + h) % NPH], sem_u).wait()
            pltpu.make_async_copy(iemb_hbm.at[pl.ds(0, HD), pl.ds(0, 128)],
                                  ibufs.at[(ph + h) % NPH], sem_i).wait()

    c0, osl0 = load_ids(0)
    for w in range(DEPTH):
        fire(uid_v[c0, osl0], iid_v[c0, osl0], w, (2 * w) % NPH)

    def sg_step(sg, carry):
        c, osl = load_ids(sg)
        vec_u = uid_v[c, osl]
        vec_i = iid_v[c, osl]
        sgn = jnp.minimum(sg + 1, NSG - 1)
        cn, osln = load_ids(sgn)
        vec_un = uid_v[cn, osln]
        vec_in = iid_v[cn, osln]

        acc = zero16

        for w in range(L):
            ph = (2 * w) % NPH
            nxt = w + DEPTH
            if nxt < L:
                fire(vec_u, vec_i, nxt, (2 * nxt) % NPH)
            else:
                fire(vec_un, vec_in, nxt - L, (2 * nxt) % NPH)
            drain(ph)
            cu = jnp.full((L,), vec_u[w] & 127, jnp.int32)
            ci = jnp.full((L,), vec_i[w] & 127, jnp.int32)
            p = None
            for q in range(D // L):
                h = q // 2
                rows = iota16 + (q % 2) * L
                uvals = plsc.load_gather(ubufs.at[(ph + h) % NPH], [rows, cu])
                ivals = plsc.load_gather(ibufs.at[(ph + h) % NPH], [rows, ci])
                t = uvals * ivals
                p = t if p is None else p + t
            dot = lax.reduce_sum_p.bind(p, axes=(0,))
            acc = jnp.where(iota16 == w, acc + dot, acc)
        out_v[c, osl] = acc
        return carry

    lax.fori_loop(0, NSG, sg_step, 0)

    for w in range(DEPTH):
        drain((2 * w) % NPH)

    for c in range(NCHUNK):
        pltpu.sync_copy(out_v.at[c], out_hbm.at[pl.ds(base + c * CHUNK, CHUNK)])


def _bias_body(part_hbm, uid_hbm, iid_hbm, ubw_hbm, ibw_hbm, gb_hbm,
               out_hbm,
               uid_v, iid_v, part_v, ub_v, ib_v, gb_v, out_v,
               sem_ub, sem_ib):
    wid = lax.axis_index("s") * NC + lax.axis_index("c")
    base = wid * BPW

    for c in range(NCHUNK):
        src = pl.ds(base + c * CHUNK, CHUNK)
        pltpu.sync_copy(uid_hbm.at[src], uid_v.at[c])
        pltpu.sync_copy(iid_hbm.at[src], iid_v.at[c])
        pltpu.sync_copy(part_hbm.at[src], part_v.at[c])
    pltpu.sync_copy(gb_hbm, gb_v.at[pl.ds(0, 1)])

    copies = []
    for c in range(NCHUNK):
        rsl = pl.ds(c * CHUNK, CHUNK)
        copies.append(pltpu.async_copy(ubw_hbm.at[uid_v.at[c]],
                                       ub_v.at[rsl], sem_ub))
        copies.append(pltpu.async_copy(ibw_hbm.at[iid_v.at[c]],
                                       ib_v.at[rsl], sem_ib))
    gb = gb_v[pl.ds(0, L)][0]
    for cp in copies:
        cp.wait()

    for c in range(NCHUNK):
        for g in range(CHUNK // L):
            gsl = pl.ds(g * L, L)
            asl = pl.ds(c * CHUNK + g * L, L)
            out_v[c, gsl] = part_v[c, gsl] + ub_v[asl] + ib_v[asl] + gb

    for c in range(NCHUNK):
        pltpu.sync_copy(out_v.at[c], out_hbm.at[pl.ds(base + c * CHUNK, CHUNK)])


@jax.jit
def _mf_predict(user_ids, item_ids, uemb_t, iemb_t,
                user_bias_w, item_bias_w, global_bias):
    mesh = plsc.VectorSubcoreMesh(core_axis_name="c", subcore_axis_name="s",
                                  num_cores=NC, num_subcores=NS)
    kfn = pl.kernel(
        _body,
        out_type=jax.ShapeDtypeStruct((B,), jnp.float32),
        mesh=mesh,
        scratch_types=[
            pltpu.VMEM((NCHUNK, CHUNK), jnp.int32),
            pltpu.VMEM((NCHUNK, CHUNK), jnp.int32),
            pltpu.VMEM((NPH, HD, 128), jnp.float32),
            pltpu.VMEM((NPH, HD, 128), jnp.float32),
            pltpu.VMEM((NCHUNK, CHUNK), jnp.float32),
            pltpu.SemaphoreType.DMA,
            pltpu.SemaphoreType.DMA,
        ],
        compiler_params=pltpu.CompilerParams(needs_layout_passes=False,
                                             use_tc_tiling_on_sc=True),
    )
    part = kfn(user_ids, item_ids, uemb_t, iemb_t)
    bfn = pl.kernel(
        _bias_body,
        out_type=jax.ShapeDtypeStruct((B,), jnp.float32),
        mesh=mesh,
        scratch_types=[
            pltpu.VMEM((NCHUNK, CHUNK), jnp.int32),
            pltpu.VMEM((NCHUNK, CHUNK), jnp.int32),
            pltpu.VMEM((NCHUNK, CHUNK), jnp.float32),
            pltpu.VMEM((BPW,), jnp.float32),
            pltpu.VMEM((BPW,), jnp.float32),
            pltpu.VMEM((L,), jnp.float32),
            pltpu.VMEM((NCHUNK, CHUNK), jnp.float32),
            pltpu.SemaphoreType.DMA,
            pltpu.SemaphoreType.DMA,
        ],
        compiler_params=pltpu.CompilerParams(needs_layout_passes=False),
    )
    return bfn(part, user_ids, item_ids,
               user_bias_w, item_bias_w, global_bias)


def kernel(user_ids, item_ids, user_emb, item_emb, user_bias_w, item_bias_w,
           global_bias):
    return _mf_predict(user_ids.astype(jnp.int32), item_ids.astype(jnp.int32),
                       user_emb.T, item_emb.T,
                       user_bias_w.reshape(-1), item_bias_w.reshape(-1),
                       global_bias)


# speedup vs baseline: 1.0000x; 1.0000x over previous
import jax
import jax.numpy as jnp
from jax import lax
from jax.experimental import pallas as pl
from jax.experimental.pallas import tpu as pltpu
from jax.experimental.pallas import tpu_sc as plsc

B = 16384
D = 64
NC = 2
NS = 16
L = 16
NW = NC * NS
BPW = B // NW
CHUNK = 128
NCHUNK = BPW // CHUNK
NPH = 8
DEPTH = 3
HD = D // 2
NSG = BPW // L


def _body(uid_hbm, iid_hbm, uemb_hbm, iemb_hbm,
          out_hbm,
          uid_v, iid_v, ubufs, ibufs, out_v, sem_u, sem_i):
    wid = lax.axis_index("s") * NC + lax.axis_index("c")
    base = wid * BPW

    for c in range(NCHUNK):
        src = pl.ds(base + c * CHUNK, CHUNK)
        pltpu.sync_copy(uid_hbm.at[src], uid_v.at[c])
        pltpu.sync_copy(iid_hbm.at[src], iid_v.at[c])

    iota16 = lax.iota(jnp.int32, L)
    zero16 = jnp.zeros((L,), jnp.float32)

    def load_ids(sg):
        c = sg // (CHUNK // L)
        off = (sg % (CHUNK // L)) * L
        return c, pl.ds(off, L)

    def fire(vec_u, vec_i, lane, ph):
        su = vec_u[lane]
        si = vec_i[lane]
        su0 = pl.multiple_of((su >> 7) * 128, 128)
        si0 = pl.multiple_of((si >> 7) * 128, 128)
        for h in range(2):
            rsl = pl.ds(h * HD, HD)
            pltpu.async_copy(uemb_hbm.at[rsl, pl.ds(su0, 128)],
                             ubufs.at[(ph + h) % NPH], sem_u)
            pltpu.async_copy(iemb_hbm.at[rsl, pl.ds(si0, 128)],
                             ibufs.at[(ph + h) % NPH], sem_i)

    def drain(ph):
        for h in range(2):
            pltpu.make_async_copy(uemb_hbm.at[pl.ds(0, HD), pl.ds(0, 128)],
                                  ubufs.at[(ph + h) % NPH], sem_u).wait()
            pltpu.make_async_copy(iemb_hbm.at[pl.ds(0, HD), pl.ds(0, 128)],
                                  ibufs.at[(ph + h) % NPH], sem_i).wait()

    c0, osl0 = load_ids(0)
    for w in range(DEPTH):
        fire(uid_v[c0, osl0], iid_v[c0, osl0], w, (2 * w) % NPH)

    def sg_step(sg, carry):
        c, osl = load_ids(sg)
        vec_u = uid_v[c, osl]
        vec_i = iid_v[c, osl]
        sgn = jnp.minimum(sg + 1, NSG - 1)
        cn, osln = load_ids(sgn)
        vec_un = uid_v[cn, osln]
        vec_in = iid_v[cn, osln]

        acc = zero16

        for w in range(L):
            ph = (2 * w) % NPH
            nxt = w + DEPTH
            if nxt < L:
                fire(vec_u, vec_i, nxt, (2 * nxt) % NPH)
            else:
                fire(vec_un, vec_in, nxt - L, (2 * nxt) % NPH)
            drain(ph)
            cu = jnp.full((L,), vec_u[w] & 127, jnp.int32)
            ci = jnp.full((L,), vec_i[w] & 127, jnp.int32)
            p = None
            for q in range(D // L):
                h = q // 2
                rows = iota16 + (q % 2) * L
                uvals = plsc.load_gather(ubufs.at[(ph + h) % NPH], [rows, cu])
                ivals = plsc.load_gather(ibufs.at[(ph + h) % NPH], [rows, ci])
                t = uvals * ivals
                p = t if p is None else p + t
            dot = lax.reduce_sum_p.bind(p, axes=(0,))
            acc = jnp.where(iota16 == w, acc + dot, acc)
        out_v[c, osl] = acc
        return carry

    lax.fori_loop(0, NSG, sg_step, 0)

    for w in range(DEPTH):
        drain((2 * w) % NPH)

    for c in range(NCHUNK):
        pltpu.sync_copy(out_v.at[c], out_hbm.at[pl.ds(base + c * CHUNK, CHUNK)])


def _bias_body(part_hbm, uid_hbm, iid_hbm, ubw_hbm, ibw_hbm, gb_hbm,
               out_hbm,
               uid_v, iid_v, part_v, ub_v, ib_v, gb_v, out_v,
               sem_ub, sem_ib):
    wid = lax.axis_index("s") * NC + lax.axis_index("c")
    base = wid * BPW

    for c in range(NCHUNK):
        src = pl.ds(base + c * CHUNK, CHUNK)
        pltpu.sync_copy(uid_hbm.at[src], uid_v.at[c])
        pltpu.sync_copy(iid_hbm.at[src], iid_v.at[c])
        pltpu.sync_copy(part_hbm.at[src], part_v.at[c])
    pltpu.sync_copy(gb_hbm, gb_v.at[pl.ds(0, 1)])

    copies = []
    for c in range(NCHUNK):
        rsl = pl.ds(c * CHUNK, CHUNK)
        copies.append(pltpu.async_copy(ubw_hbm.at[uid_v.at[c]],
                                       ub_v.at[rsl], sem_ub))
        copies.append(pltpu.async_copy(ibw_hbm.at[iid_v.at[c]],
                                       ib_v.at[rsl], sem_ib))
    gb = gb_v[pl.ds(0, L)][0]
    for cp in copies:
        cp.wait()

    for c in range(NCHUNK):
        for g in range(CHUNK // L):
            gsl = pl.ds(g * L, L)
            asl = pl.ds(c * CHUNK + g * L, L)
            out_v[c, gsl] = part_v[c, gsl] + ub_v[asl] + ib_v[asl] + gb

    for c in range(NCHUNK):
        pltpu.sync_copy(out_v.at[c], out_hbm.at[pl.ds(base + c * CHUNK, CHUNK)])


@jax.jit
def _mf_predict(user_ids, item_ids, uemb_t, iemb_t,
                user_bias_w, item_bias_w, global_bias):
    mesh = plsc.VectorSubcoreMesh(core_axis_name="c", subcore_axis_name="s",
                                  num_cores=NC, num_subcores=NS)
    kfn = pl.kernel(
        _body,
        out_type=jax.ShapeDtypeStruct((B,), jnp.float32),
        mesh=mesh,
        scratch_types=[
            pltpu.VMEM((NCHUNK, CHUNK), jnp.int32),
            pltpu.VMEM((NCHUNK, CHUNK), jnp.int32),
            pltpu.VMEM((NPH, HD, 128), jnp.float32),
            pltpu.VMEM((NPH, HD, 128), jnp.float32),
            pltpu.VMEM((NCHUNK, CHUNK), jnp.float32),
            pltpu.SemaphoreType.DMA,
            pltpu.SemaphoreType.DMA,
        ],
        compiler_params=pltpu.CompilerParams(needs_layout_passes=False,
                                             use_tc_tiling_on_sc=True),
    )
    part = kfn(user_ids, item_ids, uemb_t, iemb_t)
    bfn = pl.kernel(
        _bias_body,
        out_type=jax.ShapeDtypeStruct((B,), jnp.float32),
        mesh=mesh,
        scratch_types=[
            pltpu.VMEM((NCHUNK, CHUNK), jnp.int32),
            pltpu.VMEM((NCHUNK, CHUNK), jnp.int32),
            pltpu.VMEM((NCHUNK, CHUNK), jnp.float32),
            pltpu.VMEM((BPW,), jnp.float32),
            pltpu.VMEM((BPW,), jnp.float32),
            pltpu.VMEM((L,), jnp.float32),
            pltpu.VMEM((NCHUNK, CHUNK), jnp.float32),
            pltpu.SemaphoreType.DMA,
            pltpu.SemaphoreType.DMA,
        ],
        compiler_params=pltpu.CompilerParams(needs_layout_passes=False),
    )
    return bfn(part, user_ids, item_ids,
               user_bias_w, item_bias_w, global_bias)


def kernel(user_ids, item_ids, user_emb, item_emb, user_bias_w, item_bias_w,
           global_bias):
    return _mf_predict(user_ids.astype(jnp.int32), item_ids.astype(jnp.int32),
                       user_emb.T, item_emb.T,
                       user_bias_w.reshape(-1), item_bias_w.reshape(-1),
                       global_bias)
